# Initial kernel scaffold; baseline (speedup 1.0000x reference)
import jax
import jax.numpy as jnp
from jax.experimental import pallas as pl


def kernel(h, edge_index, bias):
    raise NotImplementedError("write your pallas kernel here")


# speedup vs baseline: 4.2881x; 4.2881x over previous
import functools

import jax
import jax.numpy as jnp
from jax import lax
from jax.experimental import pallas as pl
from jax.experimental.pallas import tpu as pltpu
from jax.experimental.pallas import tpu_sc as plsc

LANES = 16
NUM_WORKERS = 32
CHUNK = 64
NBUF = 3


def _make_scorer(n_nodes, d_feat, e_pad):
    per_tile = e_pad // NUM_WORKERS
    n_chunks = per_tile // CHUNK
    n_super = n_chunks // NBUF

    @functools.partial(
        pl.kernel,
        mesh=plsc.VectorSubcoreMesh(core_axis_name="c", subcore_axis_name="s"),
        out_type=jax.ShapeDtypeStruct((e_pad,), jnp.float32),
        compiler_params=pltpu.CompilerParams(needs_layout_passes=False),
        scratch_types=[
            pltpu.VMEM((per_tile,), jnp.int32),
            pltpu.VMEM((per_tile,), jnp.int32),
            [pltpu.VMEM((CHUNK, d_feat), jnp.float32) for _ in range(NBUF)],
            [pltpu.VMEM((CHUNK, d_feat), jnp.float32) for _ in range(NBUF)],
            [pltpu.VMEM((CHUNK,), jnp.float32) for _ in range(NBUF)],
            pltpu.VMEM((n_nodes,), jnp.float32),
            pltpu.VMEM((LANES * (LANES + 1),), jnp.float32),
            [pltpu.SemaphoreType.DMA for _ in range(NBUF)],
            [pltpu.SemaphoreType.DMA for _ in range(NBUF)],
        ],
    )
    def scorer(h_hbm, src_hbm, dst_hbm, bias_hbm, out_hbm,
               idx_s, idx_d, rows_s, rows_d, out_v, bias_v, tr_v,
               sem_g, sem_w):
        wid = lax.axis_index("s") * 2 + lax.axis_index("c")
        tile_base = wid * per_tile
        pltpu.sync_copy(bias_hbm, bias_v)
        pltpu.sync_copy(src_hbm.at[pl.ds(tile_base, per_tile)], idx_s)
        pltpu.sync_copy(dst_hbm.at[pl.ds(tile_base, per_tile)], idx_d)

        def fire(c, b):
            pltpu.async_copy(
                h_hbm.at[idx_s.at[pl.ds(c * CHUNK, CHUNK)]], rows_s[b], sem_g[b])
            pltpu.async_copy(
                h_hbm.at[idx_d.at[pl.ds(c * CHUNK, CHUNK)]], rows_d[b], sem_g[b])

        def drain_gathers(b):
            dummy = h_hbm.at[pl.ds(0, CHUNK)]
            pltpu.make_async_copy(dummy, rows_s[b], sem_g[b]).wait()
            pltpu.make_async_copy(dummy, rows_d[b], sem_g[b]).wait()

        def compute(c, b):
            rs, rd = rows_s[b], rows_d[b]

            def group_body(g, _):
                for e in range(LANES):
                    r = g * LANES + e
                    acc = rs[r, pl.ds(0, LANES)] * rd[r, pl.ds(0, LANES)]
                    for j in range(1, d_feat // LANES):
                        acc = acc + (rs[r, pl.ds(j * LANES, LANES)]
                                     * rd[r, pl.ds(j * LANES, LANES)])
                    tr_v[pl.ds(e * (LANES + 1), LANES)] = acc

                lane17 = lax.iota(jnp.int32, LANES) * (LANES + 1)
                tot = plsc.load_gather(tr_v, [lane17])
                for cc in range(1, LANES):
                    tot = tot + plsc.load_gather(tr_v, [lane17 + cc])

                e0 = c * CHUNK + g * LANES
                is16 = idx_s[pl.ds(e0, LANES)]
                id16 = idx_d[pl.ds(e0, LANES)]
                tot = tot + (plsc.load_gather(bias_v, [is16])
                             + plsc.load_gather(bias_v, [id16]))
                out_v[b][pl.ds(g * LANES, LANES)] = tot
                return 0

            lax.fori_loop(0, CHUNK // LANES, group_body, 0)

        for b in range(NBUF - 1):
            fire(b, b)

        def super_body(s, _):
            for b in range(NBUF):
                c = s * NBUF + b

                @pl.when(c >= NBUF)
                def _():
                    pltpu.make_async_copy(
                        out_v[b], out_hbm.at[pl.ds(0, CHUNK)], sem_w[b]).wait()

                drain_gathers(b)
                compute(c, b)
                pltpu.async_copy(
                    out_v[b], out_hbm.at[pl.ds(tile_base + c * CHUNK, CHUNK)],
                    sem_w[b])

                @pl.when(c + NBUF - 1 < n_chunks)
                def _():
                    fire(c + NBUF - 1, (b - 1) % NBUF)
            return 0

        lax.fori_loop(0, n_super, super_body, 0)
        for b in range(NBUF):
            pltpu.make_async_copy(
                out_v[b], out_hbm.at[pl.ds(0, CHUNK)], sem_w[b]).wait()

    return scorer


def kernel(h, edge_index, bias):
    n_nodes, d_feat = h.shape
    e = edge_index.shape[1]
    src = edge_index[0].astype(jnp.int32)
    dst = edge_index[1].astype(jnp.int32)

    step = NUM_WORKERS * CHUNK * NBUF
    e_pad = ((e + step - 1) // step) * step
    pad = e_pad - e
    if pad:
        src = jnp.pad(src, (0, pad))
        dst = jnp.pad(dst, (0, pad))

    scorer = _make_scorer(n_nodes, d_feat, e_pad)
    out = scorer(h, src, dst, bias.astype(jnp.float32))
    return out[:e]


# speedup vs baseline: 5.1666x; 1.2048x over previous
import functools

import jax
import jax.numpy as jnp
from jax import lax
from jax.experimental import pallas as pl
from jax.experimental.pallas import tpu as pltpu
from jax.experimental.pallas import tpu_sc as plsc

LANES = 16
NUM_WORKERS = 32
CHUNK = 128
NBUF = 2


def _make_scorer(n_nodes, d_feat, e_pad):
    per_tile = e_pad // NUM_WORKERS
    n_chunks = per_tile // CHUNK
    n_super = n_chunks // NBUF

    @functools.partial(
        pl.kernel,
        mesh=plsc.VectorSubcoreMesh(core_axis_name="c", subcore_axis_name="s"),
        out_type=jax.ShapeDtypeStruct((e_pad,), jnp.float32),
        compiler_params=pltpu.CompilerParams(needs_layout_passes=False),
        scratch_types=[
            pltpu.VMEM((per_tile,), jnp.int32),
            pltpu.VMEM((per_tile,), jnp.int32),
            [pltpu.VMEM((CHUNK, d_feat // 2), jnp.int32) for _ in range(NBUF)],
            [pltpu.VMEM((CHUNK, d_feat // 2), jnp.int32) for _ in range(NBUF)],
            [pltpu.VMEM((CHUNK,), jnp.float32) for _ in range(NBUF)],
            pltpu.VMEM((n_nodes,), jnp.float32),
            pltpu.VMEM((LANES * (LANES + 1),), jnp.float32),
            [pltpu.SemaphoreType.DMA for _ in range(NBUF)],
            [pltpu.SemaphoreType.DMA for _ in range(NBUF)],
        ],
    )
    def scorer(h_hbm, src_hbm, dst_hbm, bias_hbm, out_hbm,
               idx_s, idx_d, rows_s, rows_d, out_v, bias_v, tr_v,
               sem_g, sem_w):
        wid = lax.axis_index("s") * 2 + lax.axis_index("c")
        tile_base = wid * per_tile
        pltpu.sync_copy(bias_hbm, bias_v)
        pltpu.sync_copy(src_hbm.at[pl.ds(tile_base, per_tile)], idx_s)
        pltpu.sync_copy(dst_hbm.at[pl.ds(tile_base, per_tile)], idx_d)

        def fire(c, b):
            pltpu.async_copy(
                h_hbm.at[idx_s.at[pl.ds(c * CHUNK, CHUNK)]], rows_s[b], sem_g[b])
            pltpu.async_copy(
                h_hbm.at[idx_d.at[pl.ds(c * CHUNK, CHUNK)]], rows_d[b], sem_g[b])

        def drain_gathers(b):
            dummy = h_hbm.at[pl.ds(0, CHUNK)]
            pltpu.make_async_copy(dummy, rows_s[b], sem_g[b]).wait()
            pltpu.make_async_copy(dummy, rows_d[b], sem_g[b]).wait()

        def compute(c, b):
            rs, rd = rows_s[b], rows_d[b]

            def group_body(g, _):
                for e in range(LANES):
                    r = g * LANES + e
                    acc = jnp.zeros((LANES,), jnp.float32)
                    for j in range(d_feat // (2 * LANES)):
                        ps = plsc.bitcast(rs[r, pl.ds(j * LANES, LANES)],
                                          jnp.bfloat16)
                        pd = plsc.bitcast(rd[r, pl.ds(j * LANES, LANES)],
                                          jnp.bfloat16)
                        a_s, b_s = plsc.unpack(
                            ps, format=plsc.PackFormat.INTERLEAVED)
                        a_d, b_d = plsc.unpack(
                            pd, format=plsc.PackFormat.INTERLEAVED)
                        acc = acc + a_s * a_d + b_s * b_d
                    tr_v[pl.ds(e * (LANES + 1), LANES)] = acc

                lane17 = lax.iota(jnp.int32, LANES) * (LANES + 1)
                tot = plsc.load_gather(tr_v, [lane17])
                for cc in range(1, LANES):
                    tot = tot + plsc.load_gather(tr_v, [lane17 + cc])

                e0 = c * CHUNK + g * LANES
                is16 = idx_s[pl.ds(e0, LANES)]
                id16 = idx_d[pl.ds(e0, LANES)]
                tot = tot + (plsc.load_gather(bias_v, [is16])
                             + plsc.load_gather(bias_v, [id16]))
                out_v[b][pl.ds(g * LANES, LANES)] = tot
                return 0

            lax.fori_loop(0, CHUNK // LANES, group_body, 0)

        for b in range(NBUF - 1):
            fire(b, b)

        def super_body(s, _):
            for b in range(NBUF):
                c = s * NBUF + b

                @pl.when(c >= NBUF)
                def _():
                    pltpu.make_async_copy(
                        out_v[b], out_hbm.at[pl.ds(0, CHUNK)], sem_w[b]).wait()

                drain_gathers(b)
                compute(c, b)
                pltpu.async_copy(
                    out_v[b], out_hbm.at[pl.ds(tile_base + c * CHUNK, CHUNK)],
                    sem_w[b])

                @pl.when(c + NBUF - 1 < n_chunks)
                def _():
                    fire(c + NBUF - 1, (b - 1) % NBUF)
            return 0

        lax.fori_loop(0, n_super, super_body, 0)
        for b in range(NBUF):
            pltpu.make_async_copy(
                out_v[b], out_hbm.at[pl.ds(0, CHUNK)], sem_w[b]).wait()

    return scorer


def kernel(h, edge_index, bias):
    n_nodes, d_feat = h.shape
    e = edge_index.shape[1]
    src = edge_index[0].astype(jnp.int32)
    dst = edge_index[1].astype(jnp.int32)

    step = NUM_WORKERS * CHUNK * NBUF
    e_pad = ((e + step - 1) // step) * step
    pad = e_pad - e
    if pad:
        src = jnp.pad(src, (0, pad))
        dst = jnp.pad(dst, (0, pad))

    h_pk = jax.lax.bitcast_convert_type(
        h.astype(jnp.bfloat16).reshape(n_nodes, d_feat // 2, 2), jnp.int32)
    scorer = _make_scorer(n_nodes, d_feat, e_pad)
    out = scorer(h_pk, src, dst, bias.astype(jnp.float32))
    return out[:e]


# speedup vs baseline: 10.6237x; 2.0562x over previous
import functools

import jax
import jax.numpy as jnp
from jax import lax
from jax.experimental import pallas as pl
from jax.experimental.pallas import tpu as pltpu
from jax.experimental.pallas import tpu_sc as plsc

LANES = 16
NUM_WORKERS = 32
FPB = 4


def _make_scorer(n_nodes, d_feat, e_pad):
    per_tile = e_pad // NUM_WORKERS
    n_groups = per_tile // LANES
    n_blk = (d_feat // 2) // FPB
    n_super = n_blk // 2

    @functools.partial(
        pl.kernel,
        mesh=plsc.VectorSubcoreMesh(core_axis_name="c", subcore_axis_name="s"),
        out_type=jax.ShapeDtypeStruct((e_pad,), jnp.float32),
        compiler_params=pltpu.CompilerParams(needs_layout_passes=False),
        scratch_types=[
            pltpu.VMEM((per_tile,), jnp.int32),
            pltpu.VMEM((per_tile,), jnp.int32),
            [pltpu.VMEM((FPB, n_nodes), jnp.int32) for _ in range(2)],
            pltpu.VMEM((per_tile,), jnp.float32),
            pltpu.VMEM((1, n_nodes), jnp.float32),
            [pltpu.SemaphoreType.DMA for _ in range(2)],
        ],
    )
    def scorer(ht_hbm, src_hbm, dst_hbm, bias_hbm, out_hbm,
               idx_s, idx_d, cols, acc_v, bias_v, sem):
        wid = lax.axis_index("s") * 2 + lax.axis_index("c")
        tile_base = wid * per_tile
        pltpu.sync_copy(bias_hbm, bias_v)
        pltpu.sync_copy(src_hbm.at[pl.ds(tile_base, per_tile)], idx_s)
        pltpu.sync_copy(dst_hbm.at[pl.ds(tile_base, per_tile)], idx_d)

        zeros16 = jnp.zeros((LANES,), jnp.int32)

        def init_body(g, _):
            is16 = idx_s[pl.ds(g * LANES, LANES)]
            id16 = idx_d[pl.ds(g * LANES, LANES)]
            acc_v[pl.ds(g * LANES, LANES)] = (
                plsc.load_gather(bias_v, [zeros16, is16])
                + plsc.load_gather(bias_v, [zeros16, id16]))
            return 0

        lax.fori_loop(0, n_groups, init_body, 0)

        def fire(blk, b):
            pltpu.async_copy(ht_hbm.at[blk], cols[b], sem[b])

        def compute(b):
            col = cols[b]

            def group_body(g, _):
                is16 = idx_s[pl.ds(g * LANES, LANES)]
                id16 = idx_d[pl.ds(g * LANES, LANES)]
                acc = acc_v[pl.ds(g * LANES, LANES)]
                for p in range(FPB):
                    p16 = zeros16 + p
                    ws = plsc.load_gather(col, [p16, is16])
                    wd = plsc.load_gather(col, [p16, id16])
                    a_s, b_s = plsc.unpack(
                        plsc.bitcast(ws, jnp.bfloat16),
                        format=plsc.PackFormat.INTERLEAVED)
                    a_d, b_d = plsc.unpack(
                        plsc.bitcast(wd, jnp.bfloat16),
                        format=plsc.PackFormat.INTERLEAVED)
                    acc = acc + a_s * a_d + b_s * b_d
                acc_v[pl.ds(g * LANES, LANES)] = acc
                return 0

            lax.fori_loop(0, n_groups, group_body, 0)

        fire(0, 0)

        def super_body(s, _):
            for b in range(2):
                blk = 2 * s + b
                pltpu.make_async_copy(ht_hbm.at[0], cols[b], sem[b]).wait()

                @pl.when(blk + 1 < n_blk)
                def _():
                    fire(blk + 1, 1 - b)

                compute(b)
            return 0

        lax.fori_loop(0, n_super, super_body, 0)
        pltpu.sync_copy(acc_v, out_hbm.at[pl.ds(tile_base, per_tile)])

    return scorer


def kernel(h, edge_index, bias):
    n_nodes, d_feat = h.shape
    e = edge_index.shape[1]
    src = edge_index[0].astype(jnp.int32)
    dst = edge_index[1].astype(jnp.int32)

    step = NUM_WORKERS * LANES
    e_pad = ((e + step - 1) // step) * step
    pad = e_pad - e
    if pad:
        src = jnp.pad(src, (0, pad))
        dst = jnp.pad(dst, (0, pad))

    h_pk = jax.lax.bitcast_convert_type(
        h.astype(jnp.bfloat16).reshape(n_nodes, d_feat // 2, 2), jnp.int32)
    ht = h_pk.T.reshape(d_feat // 2 // FPB, FPB, n_nodes)

    scorer = _make_scorer(n_nodes, d_feat, e_pad)
    out = scorer(ht, src, dst, bias.reshape(1, n_nodes).astype(jnp.float32))
    return out[:e]


# speedup vs baseline: 14.1818x; 1.3349x over previous
import functools

import jax
import jax.numpy as jnp
from jax import lax
from jax.experimental import pallas as pl
from jax.experimental.pallas import tpu as pltpu
from jax.experimental.pallas import tpu_sc as plsc

LANES = 16
NUM_WORKERS = 32
FPB = 4


def _make_scorer(n_nodes, d_feat, e_pad):
    per_tile = e_pad // NUM_WORKERS
    n_groups = per_tile // LANES
    n_blk = (d_feat // 2) // FPB
    n_super = n_blk // 2

    @functools.partial(
        pl.kernel,
        mesh=plsc.VectorSubcoreMesh(core_axis_name="c", subcore_axis_name="s"),
        out_type=jax.ShapeDtypeStruct((e_pad,), jnp.float32),
        compiler_params=pltpu.CompilerParams(needs_layout_passes=False),
        scratch_types=[
            pltpu.VMEM((per_tile,), jnp.int32),
            pltpu.VMEM((per_tile,), jnp.int32),
            [pltpu.VMEM((FPB, n_nodes), jnp.int32) for _ in range(2)],
            pltpu.VMEM((per_tile,), jnp.float32),
            pltpu.VMEM((1, n_nodes), jnp.float32),
            [pltpu.SemaphoreType.DMA for _ in range(2)],
        ],
    )
    def scorer(ht_hbm, src_hbm, dst_hbm, bias_hbm, out_hbm,
               idx_s, idx_d, cols, acc_v, bias_v, sem):
        wid = lax.axis_index("s") * 2 + lax.axis_index("c")
        tile_base = wid * per_tile
        pltpu.sync_copy(bias_hbm, bias_v)
        pltpu.sync_copy(src_hbm.at[pl.ds(tile_base, per_tile)], idx_s)
        pltpu.sync_copy(dst_hbm.at[pl.ds(tile_base, per_tile)], idx_d)

        zeros16 = jnp.zeros((LANES,), jnp.int32)

        def init_body(g, _):
            is16 = idx_s[pl.ds(g * LANES, LANES)]
            id16 = idx_d[pl.ds(g * LANES, LANES)]
            acc_v[pl.ds(g * LANES, LANES)] = (
                plsc.load_gather(bias_v, [zeros16, is16])
                + plsc.load_gather(bias_v, [zeros16, id16]))
            return 0

        lax.fori_loop(0, n_groups, init_body, 0)

        def fire(blk, b):
            pltpu.async_copy(ht_hbm.at[blk], cols[b], sem[b])

        def compute(b):
            col = cols[b]

            def group_body(g, _):
                is16 = idx_s[pl.ds(g * LANES, LANES)]
                id16 = idx_d[pl.ds(g * LANES, LANES)]
                acc = acc_v[pl.ds(g * LANES, LANES)]
                for p in range(FPB):
                    p16 = zeros16 + p
                    ws = plsc.load_gather(col, [p16, is16])
                    wd = plsc.load_gather(col, [p16, id16])
                    a_s, b_s = plsc.unpack(
                        plsc.bitcast(ws, jnp.bfloat16),
                        format=plsc.PackFormat.INTERLEAVED)
                    a_d, b_d = plsc.unpack(
                        plsc.bitcast(wd, jnp.bfloat16),
                        format=plsc.PackFormat.INTERLEAVED)
                    acc = acc + a_s * a_d + b_s * b_d
                acc_v[pl.ds(g * LANES, LANES)] = acc
                return 0

            lax.fori_loop(0, n_groups, group_body, 0)

        fire(0, 0)

        def super_body(s, _):
            for b in range(2):
                blk = 2 * s + b
                pltpu.make_async_copy(ht_hbm.at[0], cols[b], sem[b]).wait()

                @pl.when(blk + 1 < n_blk)
                def _():
                    fire(blk + 1, 1 - b)

                compute(b)
            return 0

        lax.fori_loop(0, n_super, super_body, 0)
        pltpu.sync_copy(acc_v, out_hbm.at[pl.ds(tile_base, per_tile)])

    return scorer


def kernel(h, edge_index, bias):
    n_nodes, d_feat = h.shape
    e = edge_index.shape[1]
    src = edge_index[0].astype(jnp.int32)
    dst = edge_index[1].astype(jnp.int32)

    step = NUM_WORKERS * LANES
    e_pad = ((e + step - 1) // step) * step
    pad = e_pad - e
    if pad:
        src = jnp.pad(src, (0, pad))
        dst = jnp.pad(dst, (0, pad))

    hd = d_feat // 2
    bits = jax.lax.bitcast_convert_type(h.astype(jnp.bfloat16).T, jnp.uint16)
    words = (bits[hd:].astype(jnp.uint32) << 16) | bits[:hd].astype(jnp.uint32)
    ht = jax.lax.bitcast_convert_type(words, jnp.int32).reshape(
        hd // FPB, FPB, n_nodes)

    scorer = _make_scorer(n_nodes, d_feat, e_pad)
    out = scorer(ht, src, dst, bias.reshape(1, n_nodes).astype(jnp.float32))
    return out[:e]


# speedup vs baseline: 19.3970x; 1.3677x over previous
import functools

import jax
import jax.numpy as jnp
from jax import lax
from jax.experimental import pallas as pl
from jax.experimental.pallas import tpu as pltpu
from jax.experimental.pallas import tpu_sc as plsc

LANES = 16
NUM_WORKERS = 32
FPB = 4


def _make_scorer(n_nodes, d_feat, e_pad):
    per_tile = e_pad // NUM_WORKERS
    n_groups = per_tile // LANES
    n_blk = (d_feat // 2) // FPB
    n_super = n_blk // 2

    @functools.partial(
        pl.kernel,
        mesh=plsc.VectorSubcoreMesh(core_axis_name="c", subcore_axis_name="s"),
        out_type=jax.ShapeDtypeStruct((e_pad,), jnp.float32),
        compiler_params=pltpu.CompilerParams(needs_layout_passes=False),
        scratch_types=[
            pltpu.VMEM((per_tile,), jnp.int32),
            pltpu.VMEM((per_tile,), jnp.int32),
            [pltpu.VMEM((FPB, n_nodes), jnp.int32) for _ in range(2)],
            pltpu.VMEM((per_tile,), jnp.float32),
            pltpu.VMEM((1, n_nodes), jnp.float32),
            [pltpu.SemaphoreType.DMA for _ in range(2)],
        ],
    )
    def scorer(ht_hbm, src_hbm, dst_hbm, bias_hbm, out_hbm,
               idx_s, idx_d, cols, acc_v, bias_v, sem):
        wid = lax.axis_index("s") * 2 + lax.axis_index("c")
        tile_base = wid * per_tile
        pltpu.sync_copy(bias_hbm, bias_v)
        pltpu.sync_copy(src_hbm.at[pl.ds(tile_base, per_tile)], idx_s)
        pltpu.sync_copy(dst_hbm.at[pl.ds(tile_base, per_tile)], idx_d)

        zeros16 = jnp.zeros((LANES,), jnp.int32)

        @plsc.parallel_loop(0, n_groups, unroll=2)
        def init_body(g):
            is16 = idx_s[pl.ds(g * LANES, LANES)]
            id16 = idx_d[pl.ds(g * LANES, LANES)]
            acc_v[pl.ds(g * LANES, LANES)] = (
                plsc.load_gather(bias_v, [zeros16, is16])
                + plsc.load_gather(bias_v, [zeros16, id16]))

        def fire(blk, b):
            pltpu.async_copy(ht_hbm.at[blk], cols[b], sem[b])

        def compute(b):
            col = cols[b]

            @plsc.parallel_loop(0, n_groups, unroll=2)
            def group_body(g):
                is16 = idx_s[pl.ds(g * LANES, LANES)]
                id16 = idx_d[pl.ds(g * LANES, LANES)]
                acc = acc_v[pl.ds(g * LANES, LANES)]
                for p in range(FPB):
                    p16 = zeros16 + p
                    ws = plsc.load_gather(col, [p16, is16])
                    wd = plsc.load_gather(col, [p16, id16])
                    a_s, b_s = plsc.unpack(
                        plsc.bitcast(ws, jnp.bfloat16),
                        format=plsc.PackFormat.INTERLEAVED)
                    a_d, b_d = plsc.unpack(
                        plsc.bitcast(wd, jnp.bfloat16),
                        format=plsc.PackFormat.INTERLEAVED)
                    acc = acc + a_s * a_d + b_s * b_d
                acc_v[pl.ds(g * LANES, LANES)] = acc

        fire(0, 0)

        def super_body(s, _):
            for b in range(2):
                blk = 2 * s + b
                pltpu.make_async_copy(ht_hbm.at[0], cols[b], sem[b]).wait()

                @pl.when(blk + 1 < n_blk)
                def _():
                    fire(blk + 1, 1 - b)

                compute(b)
            return 0

        lax.fori_loop(0, n_super, super_body, 0)
        pltpu.sync_copy(acc_v, out_hbm.at[pl.ds(tile_base, per_tile)])

    return scorer


def kernel(h, edge_index, bias):
    n_nodes, d_feat = h.shape
    e = edge_index.shape[1]
    src = edge_index[0].astype(jnp.int32)
    dst = edge_index[1].astype(jnp.int32)

    step = NUM_WORKERS * LANES
    e_pad = ((e + step - 1) // step) * step
    pad = e_pad - e
    if pad:
        src = jnp.pad(src, (0, pad))
        dst = jnp.pad(dst, (0, pad))

    hd = d_feat // 2
    bits = jax.lax.bitcast_convert_type(h.astype(jnp.bfloat16).T, jnp.uint16)
    words = (bits[hd:].astype(jnp.uint32) << 16) | bits[:hd].astype(jnp.uint32)
    ht = jax.lax.bitcast_convert_type(words, jnp.int32).reshape(
        hd // FPB, FPB, n_nodes)

    scorer = _make_scorer(n_nodes, d_feat, e_pad)
    out = scorer(ht, src, dst, bias.reshape(1, n_nodes).astype(jnp.float32))
    return out[:e]


# speedup vs baseline: 19.9989x; 1.0310x over previous
import functools

import jax
import jax.numpy as jnp
from jax import lax
from jax.experimental import pallas as pl
from jax.experimental.pallas import tpu as pltpu
from jax.experimental.pallas import tpu_sc as plsc

LANES = 16
NUM_WORKERS = 32
FPB = 4


def _make_scorer(n_nodes, d_feat, e_pad):
    per_tile = e_pad // NUM_WORKERS
    n_groups = per_tile // LANES
    n_blk = (d_feat // 2) // FPB
    n_super = n_blk // 2

    @functools.partial(
        pl.kernel,
        mesh=plsc.VectorSubcoreMesh(core_axis_name="c", subcore_axis_name="s"),
        out_type=jax.ShapeDtypeStruct((e_pad,), jnp.float32),
        compiler_params=pltpu.CompilerParams(needs_layout_passes=False),
        scratch_types=[
            pltpu.VMEM((per_tile,), jnp.int32),
            pltpu.VMEM((per_tile,), jnp.int32),
            [pltpu.VMEM((FPB, n_nodes), jnp.int32) for _ in range(2)],
            pltpu.VMEM((per_tile,), jnp.float32),
            pltpu.VMEM((1, n_nodes), jnp.float32),
            [pltpu.SemaphoreType.DMA for _ in range(2)],
        ],
    )
    def scorer(ht_hbm, src_hbm, dst_hbm, bias_hbm, out_hbm,
               idx_s, idx_d, cols, acc_v, bias_v, sem):
        wid = lax.axis_index("s") * 2 + lax.axis_index("c")
        tile_base = wid * per_tile
        pltpu.sync_copy(bias_hbm, bias_v)
        pltpu.sync_copy(src_hbm.at[pl.ds(tile_base, per_tile)], idx_s)
        pltpu.sync_copy(dst_hbm.at[pl.ds(tile_base, per_tile)], idx_d)

        zeros16 = jnp.zeros((LANES,), jnp.int32)

        @plsc.parallel_loop(0, n_groups, unroll=4)
        def init_body(g):
            is16 = idx_s[pl.ds(g * LANES, LANES)]
            id16 = idx_d[pl.ds(g * LANES, LANES)]
            acc_v[pl.ds(g * LANES, LANES)] = (
                plsc.load_gather(bias_v, [zeros16, is16])
                + plsc.load_gather(bias_v, [zeros16, id16]))

        def fire(blk, b):
            pltpu.async_copy(ht_hbm.at[blk], cols[b], sem[b])

        def compute(b):
            col = cols[b]

            @plsc.parallel_loop(0, n_groups, unroll=4)
            def group_body(g):
                is16 = idx_s[pl.ds(g * LANES, LANES)]
                id16 = idx_d[pl.ds(g * LANES, LANES)]
                acc = acc_v[pl.ds(g * LANES, LANES)]
                for p in range(FPB):
                    p16 = zeros16 + p
                    ws = plsc.load_gather(col, [p16, is16])
                    wd = plsc.load_gather(col, [p16, id16])
                    a_s, b_s = plsc.unpack(
                        plsc.bitcast(ws, jnp.bfloat16),
                        format=plsc.PackFormat.INTERLEAVED)
                    a_d, b_d = plsc.unpack(
                        plsc.bitcast(wd, jnp.bfloat16),
                        format=plsc.PackFormat.INTERLEAVED)
                    acc = acc + a_s * a_d + b_s * b_d
                acc_v[pl.ds(g * LANES, LANES)] = acc

        fire(0, 0)

        def super_body(s, _):
            for b in range(2):
                blk = 2 * s + b
                pltpu.make_async_copy(ht_hbm.at[0], cols[b], sem[b]).wait()

                @pl.when(blk + 1 < n_blk)
                def _():
                    fire(blk + 1, 1 - b)

                compute(b)
            return 0

        lax.fori_loop(0, n_super, super_body, 0)
        pltpu.sync_copy(acc_v, out_hbm.at[pl.ds(tile_base, per_tile)])

    return scorer


def kernel(h, edge_index, bias):
    n_nodes, d_feat = h.shape
    e = edge_index.shape[1]
    src = edge_index[0].astype(jnp.int32)
    dst = edge_index[1].astype(jnp.int32)

    step = NUM_WORKERS * LANES
    e_pad = ((e + step - 1) // step) * step
    pad = e_pad - e
    if pad:
        src = jnp.pad(src, (0, pad))
        dst = jnp.pad(dst, (0, pad))

    hd = d_feat // 2
    bits = jax.lax.bitcast_convert_type(h.astype(jnp.bfloat16).T, jnp.uint16)
    words = (bits[hd:].astype(jnp.uint32) << 16) | bits[:hd].astype(jnp.uint32)
    ht = jax.lax.bitcast_convert_type(words, jnp.int32).reshape(
        hd // FPB, FPB, n_nodes)

    scorer = _make_scorer(n_nodes, d_feat, e_pad)
    out = scorer(ht, src, dst, bias.reshape(1, n_nodes).astype(jnp.float32))
    return out[:e]


# speedup vs baseline: 21.2638x; 1.0633x over previous
import functools

import jax
import jax.numpy as jnp
from jax import lax
from jax.experimental import pallas as pl
from jax.experimental.pallas import tpu as pltpu
from jax.experimental.pallas import tpu_sc as plsc

LANES = 16
NUM_WORKERS = 32
FPB = 4
UNROLL = 4


def _make_scorer(n_nodes, d_feat, e):
    align = 128
    group = NUM_WORKERS * align
    per_big = ((e + group - 1) // group) * align
    last_sz = e - (NUM_WORKERS - 1) * per_big
    assert 0 < last_sz <= per_big and last_sz % LANES == 0
    big_groups = per_big // LANES
    last_groups = last_sz // LANES
    n_blk = (d_feat // 2) // FPB
    n_super = n_blk // 2

    @functools.partial(
        pl.kernel,
        mesh=plsc.VectorSubcoreMesh(core_axis_name="c", subcore_axis_name="s"),
        out_type=jax.ShapeDtypeStruct((e,), jnp.float32),
        compiler_params=pltpu.CompilerParams(needs_layout_passes=False),
        scratch_types=[
            pltpu.VMEM((per_big,), jnp.int32),
            pltpu.VMEM((per_big,), jnp.int32),
            [pltpu.VMEM((FPB, n_nodes), jnp.int32) for _ in range(2)],
            pltpu.VMEM((per_big,), jnp.float32),
            pltpu.VMEM((1, n_nodes), jnp.float32),
            [pltpu.SemaphoreType.DMA for _ in range(2)],
        ],
    )
    def scorer(ht_hbm, edge_hbm, bias_hbm, out_hbm,
               idx_s, idx_d, cols, acc_v, bias_v, sem):
        wid = lax.axis_index("s") * 2 + lax.axis_index("c")
        tile_base = wid * per_big
        is_last = wid == NUM_WORKERS - 1
        n_groups = jnp.where(is_last, last_groups, big_groups)
        pltpu.sync_copy(bias_hbm, bias_v)

        @pl.when(is_last)
        def _():
            pltpu.sync_copy(edge_hbm.at[0].at[pl.ds(tile_base, last_sz)],
                            idx_s.at[pl.ds(0, last_sz)])
            pltpu.sync_copy(edge_hbm.at[1].at[pl.ds(tile_base, last_sz)],
                            idx_d.at[pl.ds(0, last_sz)])

        @pl.when(jnp.logical_not(is_last))
        def _():
            pltpu.sync_copy(edge_hbm.at[0].at[pl.ds(tile_base, per_big)], idx_s)
            pltpu.sync_copy(edge_hbm.at[1].at[pl.ds(tile_base, per_big)], idx_d)

        zeros16 = jnp.zeros((LANES,), jnp.int32)

        @plsc.parallel_loop(0, n_groups, unroll=UNROLL)
        def init_body(g):
            is16 = idx_s[pl.ds(g * LANES, LANES)]
            id16 = idx_d[pl.ds(g * LANES, LANES)]
            acc_v[pl.ds(g * LANES, LANES)] = (
                plsc.load_gather(bias_v, [zeros16, is16])
                + plsc.load_gather(bias_v, [zeros16, id16]))

        def fire(blk, b):
            pltpu.async_copy(ht_hbm.at[blk], cols[b], sem[b])

        def compute(b):
            col = cols[b]

            @plsc.parallel_loop(0, n_groups, unroll=UNROLL)
            def group_body(g):
                is16 = idx_s[pl.ds(g * LANES, LANES)]
                id16 = idx_d[pl.ds(g * LANES, LANES)]
                acc = acc_v[pl.ds(g * LANES, LANES)]
                for p in range(FPB):
                    p16 = zeros16 + p
                    ws = plsc.load_gather(col, [p16, is16])
                    wd = plsc.load_gather(col, [p16, id16])
                    a_s, b_s = plsc.unpack(
                        plsc.bitcast(ws, jnp.bfloat16),
                        format=plsc.PackFormat.INTERLEAVED)
                    a_d, b_d = plsc.unpack(
                        plsc.bitcast(wd, jnp.bfloat16),
                        format=plsc.PackFormat.INTERLEAVED)
                    acc = acc + a_s * a_d + b_s * b_d
                acc_v[pl.ds(g * LANES, LANES)] = acc

        fire(0, 0)

        def super_body(s, _):
            for b in range(2):
                blk = 2 * s + b
                pltpu.make_async_copy(ht_hbm.at[0], cols[b], sem[b]).wait()

                @pl.when(blk + 1 < n_blk)
                def _():
                    fire(blk + 1, 1 - b)

                compute(b)
            return 0

        lax.fori_loop(0, n_super, super_body, 0)

        @pl.when(is_last)
        def _():
            pltpu.sync_copy(acc_v.at[pl.ds(0, last_sz)],
                            out_hbm.at[pl.ds(tile_base, last_sz)])

        @pl.when(jnp.logical_not(is_last))
        def _():
            pltpu.sync_copy(acc_v, out_hbm.at[pl.ds(tile_base, per_big)])

    return scorer


def kernel(h, edge_index, bias):
    n_nodes, d_feat = h.shape
    e = edge_index.shape[1]

    hd = d_feat // 2
    bits = jax.lax.bitcast_convert_type(h.astype(jnp.bfloat16).T, jnp.uint16)
    words = (bits[hd:].astype(jnp.uint32) << 16) | bits[:hd].astype(jnp.uint32)
    ht = jax.lax.bitcast_convert_type(words, jnp.int32).reshape(
        hd // FPB, FPB, n_nodes)

    scorer = _make_scorer(n_nodes, d_feat, e)
    return scorer(ht, edge_index.astype(jnp.int32),
                  bias.reshape(1, n_nodes).astype(jnp.float32))
